# Initial kernel scaffold; baseline (speedup 1.0000x reference)
import jax
import jax.numpy as jnp
from jax.experimental import pallas as pl


def kernel(input, target_classes, weights):
    raise NotImplementedError("write your pallas kernel here")


# speedup vs baseline: 2.0495x; 2.0495x over previous
import functools

import jax
import jax.numpy as jnp
from jax import lax
from jax.experimental import pallas as pl
from jax.experimental.pallas import tpu as pltpu

C = 16
N = 1_000_000
BN = 8_000


def _body(x_ref, t_ref, w_ref, o_ref, acc_ref):
    i = pl.program_id(0)

    @pl.when(i == 0)
    def _init():
        acc_ref[...] = jnp.zeros((C, C), jnp.float32)

    t = t_ref[...]
    w = w_ref[...]
    x = x_ref[...]
    classes = lax.broadcasted_iota(jnp.int32, (BN, C), 1)
    oh = jnp.where(t == classes, w, 0.0)
    acc_ref[...] += lax.dot_general(
        oh, x, (((0,), (0,)), ((), ())), preferred_element_type=jnp.float32
    )

    @pl.when(i == pl.num_programs(0) - 1)
    def _epilogue():
        conf = acc_ref[...]
        eps = jnp.float32(jnp.finfo(jnp.float32).eps)
        conf = conf * ((1.0 - eps * C) / jnp.sum(conf))
        r = lax.broadcasted_iota(jnp.int32, (C, C), 0)
        c = lax.broadcasted_iota(jnp.int32, (C, C), 1)
        eye = (r == c).astype(jnp.float32)
        conf = conf + eye * eps
        cols = jnp.sum(conf, axis=0, keepdims=True)
        rows = jnp.sum(conf.T, axis=0, keepdims=True)
        diag = jnp.sum(conf * eye, axis=0, keepdims=True)
        kap = 2.0 * (diag - cols * rows) / (cols + rows - 2.0 * cols * rows)
        tr = jnp.float32(0.066)
        kap = jnp.where(kap >= tr, kap, tr * (jnp.tanh((kap - tr) / tr) + 1.0))
        kap = jnp.where(kap > 0, kap, eps)
        o_ref[0, 0] = 1.0 - jnp.exp(jnp.sum(jnp.log(kap)) / C)


@jax.jit
def kernel(input, target_classes, weights):
    t2 = target_classes.reshape(N, 1)
    w2 = weights.reshape(N, 1)
    out = pl.pallas_call(
        _body,
        grid=(N // BN,),
        in_specs=[
            pl.BlockSpec((BN, C), lambda i: (i, 0)),
            pl.BlockSpec((BN, 1), lambda i: (i, 0)),
            pl.BlockSpec((BN, 1), lambda i: (i, 0)),
        ],
        out_specs=pl.BlockSpec(
            block_shape=(1, 1), index_map=lambda i: (0, 0),
            memory_space=pltpu.SMEM,
        ),
        out_shape=jax.ShapeDtypeStruct((1, 1), jnp.float32),
        scratch_shapes=[pltpu.VMEM((C, C), jnp.float32)],
    )(input, t2, w2)
    return out.reshape(())


# speedup vs baseline: 5.5661x; 2.7158x over previous
import functools

import jax
import jax.numpy as jnp
from jax import lax
from jax.experimental import pallas as pl
from jax.experimental.pallas import tpu as pltpu
from jax.experimental.pallas import tpu_sc as plsc

C = 16
N = 1_000_000
R = 2_000
NCHUNKS = N // R
NW = 32
BLK = R // 16
MAX_PAIRS = (NCHUNKS // NW + 2) // 2


def _sc_body(x_hbm, t_hbm, w_hbm, out_hbm,
             xb0, xb1, tb0, tb1, wb0, wb1, conf_v, acc_v,
             sx0, sx1, st0, st1, sw0, sw1):
    cid = lax.axis_index("c")
    sid = lax.axis_index("s")
    wid = sid * 2 + cid
    nchunks = jnp.int32(NCHUNKS // NW) + (wid < NCHUNKS % NW).astype(jnp.int32)

    xb = (xb0, xb1)
    tb = (tb0, tb1)
    wb = (wb0, wb1)
    sx = (sx0, sx1)
    st = (st0, st1)
    sw = (sw0, sw1)

    zeros16 = jnp.zeros((C,), jnp.float32)
    iota16 = lax.iota(jnp.int32, C)

    for q in range(C):
        conf_v[pl.ds(q * C, C)] = zeros16

    def issue(b, ck):
        start = ck * R
        pltpu.async_copy(x_hbm.at[pl.ds(start * C, R * C)], xb[b], sx[b])
        pltpu.async_copy(t_hbm.at[pl.ds(start, R)], tb[b], st[b])
        pltpu.async_copy(w_hbm.at[pl.ds(start, R)], wb[b], sw[b])

    def wait(b):
        pltpu.make_async_copy(x_hbm.at[pl.ds(0, R * C)], xb[b], sx[b]).wait()
        pltpu.make_async_copy(t_hbm.at[pl.ds(0, R)], tb[b], st[b]).wait()
        pltpu.make_async_copy(w_hbm.at[pl.ds(0, R)], wb[b], sw[b]).wait()

    def flush(a, ct):
        plsc.addupdate_scatter(conf_v, [ct * C + iota16], a)

    def process(b):
        t_ref, w_ref, x_ref = tb[b], wb[b], xb[b]

        for q in range(4):
            acc_v[pl.ds(q * C, C)] = zeros16

        def block(i, ct):
            base = i * C
            tvec = t_ref[pl.ds(base, C)]
            wvec = w_ref[pl.ds(base, C)]
            same = jnp.logical_and(tvec[0] == ct, tvec[15] == ct)

            def fast(ct):
                accs = [acc_v[pl.ds(q * C, C)] for q in range(4)]
                for r in range(C):
                    ws = wvec[r]
                    row = x_ref[pl.ds((base + r) * C, C)]
                    accs[r % 4] = accs[r % 4] + ws * row
                for q in range(4):
                    acc_v[pl.ds(q * C, C)] = accs[q]
                return ct

            def slow(ct):
                a = acc_v[pl.ds(0, C)] + acc_v[pl.ds(C, C)] + acc_v[pl.ds(2 * C, C)] + acc_v[pl.ds(3 * C, C)]
                for r in range(C):
                    tr = tvec[r]
                    ws = wvec[r]
                    row = x_ref[pl.ds((base + r) * C, C)]
                    changed = tr != ct
                    flush(jnp.where(changed, a, 0.0), ct)
                    a = jnp.where(changed, zeros16, a) + ws * row
                    ct = jnp.where(changed, tr, ct)
                acc_v[pl.ds(0, C)] = a
                for q in range(1, 4):
                    acc_v[pl.ds(q * C, C)] = zeros16
                return ct

            return lax.cond(same, fast, slow, ct)

        ct0 = t_ref[pl.ds(0, C)][0]
        ct = lax.fori_loop(0, BLK, block, ct0)
        flush(acc_v[pl.ds(0, C)] + acc_v[pl.ds(C, C)] + acc_v[pl.ds(2 * C, C)] + acc_v[pl.ds(3 * C, C)], ct)

    issue(0, wid)
    issue(1, wid + NW)

    def pair(p, _):
        for b in range(2):
            k = 2 * p + b

            @pl.when(k < nchunks)
            def _():
                wait(b)
                process(b)

                @pl.when(k + 2 < nchunks)
                def _():
                    issue(b, wid + NW * (k + 2))

        return 0

    lax.fori_loop(0, MAX_PAIRS, pair, 0)
    pltpu.sync_copy(conf_v, out_hbm.at[pl.ds(wid * C * C, C * C)])


def _sc_confusion(x_flat, t, w):
    mesh = plsc.VectorSubcoreMesh(core_axis_name="c", subcore_axis_name="s")
    return pl.kernel(
        _sc_body,
        out_type=jax.ShapeDtypeStruct((NW * C * C,), jnp.float32),
        mesh=mesh,
        compiler_params=pltpu.CompilerParams(needs_layout_passes=False),
        scratch_types=[
            pltpu.VMEM((R * C,), jnp.float32),
            pltpu.VMEM((R * C,), jnp.float32),
            pltpu.VMEM((R,), jnp.int32),
            pltpu.VMEM((R,), jnp.int32),
            pltpu.VMEM((R,), jnp.float32),
            pltpu.VMEM((R,), jnp.float32),
            pltpu.VMEM((C * C,), jnp.float32),
            pltpu.VMEM((4 * C,), jnp.float32),
            pltpu.SemaphoreType.DMA,
            pltpu.SemaphoreType.DMA,
            pltpu.SemaphoreType.DMA,
            pltpu.SemaphoreType.DMA,
            pltpu.SemaphoreType.DMA,
            pltpu.SemaphoreType.DMA,
        ],
    )(x_flat, t, w)


def _epi_body(p_ref, o_ref):
    conf = jnp.sum(p_ref[...], axis=0)
    eps = jnp.float32(jnp.finfo(jnp.float32).eps)
    conf = conf * ((1.0 - eps * C) / jnp.sum(conf))
    r = lax.broadcasted_iota(jnp.int32, (C, C), 0)
    c = lax.broadcasted_iota(jnp.int32, (C, C), 1)
    eye = (r == c).astype(jnp.float32)
    conf = conf + eye * eps
    cols = jnp.sum(conf, axis=0, keepdims=True)
    rows = jnp.sum(conf.T, axis=0, keepdims=True)
    diag = jnp.sum(conf * eye, axis=0, keepdims=True)
    kap = 2.0 * (diag - cols * rows) / (cols + rows - 2.0 * cols * rows)
    tr = jnp.float32(0.066)
    kap = jnp.where(kap >= tr, kap, tr * (jnp.tanh((kap - tr) / tr) + 1.0))
    kap = jnp.where(kap > 0, kap, eps)
    o_ref[0, 0] = 1.0 - jnp.exp(jnp.sum(jnp.log(kap)) / C)


def _epilogue(partials):
    return pl.pallas_call(
        _epi_body,
        out_specs=pl.BlockSpec(memory_space=pltpu.SMEM),
        out_shape=jax.ShapeDtypeStruct((1, 1), jnp.float32),
    )(partials)


@jax.jit
def kernel(input, target_classes, weights):
    partials = _sc_confusion(input.reshape(-1), target_classes, weights)
    return _epilogue(partials.reshape(NW, C, C)).reshape(())


# speedup vs baseline: 49.8195x; 8.9505x over previous
import functools

import jax
import jax.numpy as jnp
from jax import lax
from jax.experimental import pallas as pl
from jax.experimental.pallas import tpu as pltpu
from jax.experimental.pallas import tpu_sc as plsc

C = 16
N = 1_000_000
R = 2_048
NCHUNKS = N // R
TAIL = N - NCHUNKS * R
NW = 32
BLK = R // 16
MAX_PAIRS = (NCHUNKS // NW + 2) // 2
CONF_WORDS = C * C * C
TAIL_BLK = 1024


def _sc_body(x_hbm, t_hbm, w_hbm, out_hbm,
             xb0, xb1, tb0, tb1, wb0, wb1, conf_v,
             sx0, sx1, st0, st1, sw0, sw1):
    cid = lax.axis_index("c")
    sid = lax.axis_index("s")
    wid = sid * 2 + cid
    nchunks = jnp.int32(NCHUNKS // NW) + (wid < NCHUNKS % NW).astype(jnp.int32)

    xb = (xb0, xb1)
    tb = (tb0, tb1)
    wb = (wb0, wb1)
    sx = (sx0, sx1)
    st = (st0, st1)
    sw = (sw0, sw1)

    zeros16 = jnp.zeros((C,), jnp.float32)
    iota16 = lax.iota(jnp.int32, C)

    for q in range(CONF_WORDS // C):
        conf_v[pl.ds(q * C, C)] = zeros16

    def issue(b, ck):
        start = ck * R
        pltpu.async_copy(x_hbm.at[:, pl.ds(start, R)], xb[b], sx[b])
        pltpu.async_copy(t_hbm.at[pl.ds(start, R)], tb[b], st[b])
        pltpu.async_copy(w_hbm.at[pl.ds(start, R)], wb[b], sw[b])

    def wait(b):
        pltpu.make_async_copy(x_hbm.at[:, pl.ds(0, R)], xb[b], sx[b]).wait()
        pltpu.make_async_copy(t_hbm.at[pl.ds(0, R)], tb[b], st[b]).wait()
        pltpu.make_async_copy(w_hbm.at[pl.ds(0, R)], wb[b], sw[b]).wait()

    def flush(psums, ct):
        for j in range(C):
            plsc.addupdate_scatter(
                conf_v, [ct * (C * C) + j * C + iota16], psums[j])

    def process(b):
        t_ref, w_ref, x_ref = tb[b], wb[b], xb[b]
        first = t_ref[pl.ds(0, C)][0]
        last = t_ref[pl.ds(R - C, C)][15]

        @pl.when(first == last)
        def _uniform_chunk():
            def group(i, psums):
                base = i * C
                wvec = w_ref[pl.ds(base, C)]
                return tuple(
                    psums[j] + wvec * x_ref[j, pl.ds(base, C)]
                    for j in range(C))

            psums = lax.fori_loop(0, BLK, group, (zeros16,) * C)
            flush(list(psums), first)

        @pl.when(first != last)
        def _boundary_chunk():
            def group(i, carry):
                ct = carry[-1]
                psums = list(carry[:C])
                base = i * C
                tvec = t_ref[pl.ds(base, C)]
                wvec = w_ref[pl.ds(base, C)]
                wm = jnp.where(tvec == ct, wvec, 0.0)
                xvs = [x_ref[j, pl.ds(base, C)] for j in range(C)]
                psums = [psums[j] + wm * xvs[j] for j in range(C)]

                def boundary(ct):
                    flush(psums, ct)
                    wleft = wvec - wm
                    for j in range(C):
                        plsc.addupdate_scatter(
                            conf_v,
                            [tvec * (C * C) + j * C + iota16],
                            wleft * xvs[j])
                    return tvec[15], jnp.float32(0.0)

                def same(ct):
                    return ct, jnp.float32(1.0)

                ct, keep = lax.cond(tvec[15] != ct, boundary, same, ct)
                return (*[p * keep for p in psums], ct)

            carry = lax.fori_loop(0, BLK, group, ((zeros16,) * C) + (first,))
            flush(list(carry[:C]), carry[-1])

    issue(0, wid)
    issue(1, wid + NW)

    def pair(p, _):
        for b in range(2):
            k = 2 * p + b

            @pl.when(k < nchunks)
            def _():
                wait(b)
                process(b)

                @pl.when(k + 2 < nchunks)
                def _():
                    issue(b, wid + NW * (k + 2))

        return 0

    lax.fori_loop(0, MAX_PAIRS, pair, 0)
    pltpu.sync_copy(conf_v, out_hbm.at[pl.ds(wid * CONF_WORDS, CONF_WORDS)])


def _sc_confusion(xT, t, w):
    mesh = plsc.VectorSubcoreMesh(core_axis_name="c", subcore_axis_name="s")
    return pl.kernel(
        _sc_body,
        out_type=jax.ShapeDtypeStruct((NW * CONF_WORDS,), jnp.float32),
        mesh=mesh,
        compiler_params=pltpu.CompilerParams(needs_layout_passes=False),
        scratch_types=[
            pltpu.VMEM((C, R), jnp.float32),
            pltpu.VMEM((C, R), jnp.float32),
            pltpu.VMEM((R,), jnp.int32),
            pltpu.VMEM((R,), jnp.int32),
            pltpu.VMEM((R,), jnp.float32),
            pltpu.VMEM((R,), jnp.float32),
            pltpu.VMEM((CONF_WORDS,), jnp.float32),
            pltpu.SemaphoreType.DMA,
            pltpu.SemaphoreType.DMA,
            pltpu.SemaphoreType.DMA,
            pltpu.SemaphoreType.DMA,
            pltpu.SemaphoreType.DMA,
            pltpu.SemaphoreType.DMA,
        ],
    )(xT, t, w)


def _epi_body(p_ref, xt_ref, t_ref, w_ref, o_ref):
    nvalid = TAIL
    t2 = t_ref[...].reshape(1, TAIL_BLK)
    w2 = w_ref[...].reshape(1, TAIL_BLK)
    lane = lax.broadcasted_iota(jnp.int32, (1, TAIL_BLK), 1)
    wv = jnp.where(lane < nvalid, w2, 0.0)
    xt = xt_ref[...]
    tail_jc = jnp.zeros((C, C), jnp.float32)
    for c in range(C):
        wc = jnp.where(t2 == c, wv, 0.0)
        s_c = jnp.sum(xt * wc, axis=1, keepdims=True)
        cl = lax.broadcasted_iota(jnp.int32, (C, C), 1)
        tail_jc = jnp.where(cl == c, s_c, tail_jc)

    conf = jnp.sum(p_ref[...], axis=(0, 3)) + tail_jc.T
    eps = jnp.float32(jnp.finfo(jnp.float32).eps)
    conf = conf * ((1.0 - eps * C) / jnp.sum(conf))
    r = lax.broadcasted_iota(jnp.int32, (C, C), 0)
    c = lax.broadcasted_iota(jnp.int32, (C, C), 1)
    eye = (r == c).astype(jnp.float32)
    conf = conf + eye * eps
    cols = jnp.sum(conf, axis=0, keepdims=True)
    rows = jnp.sum(conf.T, axis=0, keepdims=True)
    diag = jnp.sum(conf * eye, axis=0, keepdims=True)
    kap = 2.0 * (diag - cols * rows) / (cols + rows - 2.0 * cols * rows)
    tr = jnp.float32(0.066)
    kap = jnp.where(kap >= tr, kap, tr * (jnp.tanh((kap - tr) / tr) + 1.0))
    kap = jnp.where(kap > 0, kap, eps)
    o_ref[0, 0] = 1.0 - jnp.exp(jnp.sum(jnp.log(kap)) / C)


def _epilogue(partials, xT, t, w):
    tail_idx = (NCHUNKS * R) // TAIL_BLK
    return pl.pallas_call(
        _epi_body,
        grid=(1,),
        in_specs=[
            pl.BlockSpec((NW, C, C, C), lambda i: (0, 0, 0, 0)),
            pl.BlockSpec((C, TAIL_BLK), lambda i: (0, tail_idx)),
            pl.BlockSpec((TAIL_BLK,), lambda i: (tail_idx,)),
            pl.BlockSpec((TAIL_BLK,), lambda i: (tail_idx,)),
        ],
        out_specs=pl.BlockSpec(
            block_shape=(1, 1), index_map=lambda i: (0, 0),
            memory_space=pltpu.SMEM,
        ),
        out_shape=jax.ShapeDtypeStruct((1, 1), jnp.float32),
    )(partials, xT, t, w)


@jax.jit
def kernel(input, target_classes, weights):
    xT = input.T
    partials = _sc_confusion(xT, target_classes, weights)
    return _epilogue(
        partials.reshape(NW, C, C, C), xT, target_classes, weights
    ).reshape(())


# speedup vs baseline: 51.6836x; 1.0374x over previous
import functools

import jax
import jax.numpy as jnp
from jax import lax
from jax.experimental import pallas as pl
from jax.experimental.pallas import tpu as pltpu
from jax.experimental.pallas import tpu_sc as plsc

C = 16
N = 1_000_000
R = 2_048
NCHUNKS = N // R
TAIL = N - NCHUNKS * R
NW = 32
BLK = R // 16
MAX_TRIPLES = (NCHUNKS // NW + 3) // 3 + 1
UNROLL = 4
CONF_WORDS = C * C * C
TAIL_BLK = 1024


def _sc_body(x_hbm, t_hbm, w_hbm, out_hbm,
             xb0, xb1, xb2, tb0, tb1, tb2, wb0, wb1, wb2, conf_v,
             sx0, sx1, sx2, st0, st1, st2, sw0, sw1, sw2):
    cid = lax.axis_index("c")
    sid = lax.axis_index("s")
    wid = sid * 2 + cid
    nchunks = jnp.int32(NCHUNKS // NW) + (wid < NCHUNKS % NW).astype(jnp.int32)

    xb = (xb0, xb1, xb2)
    tb = (tb0, tb1, tb2)
    wb = (wb0, wb1, wb2)
    sx = (sx0, sx1, sx2)
    st = (st0, st1, st2)
    sw = (sw0, sw1, sw2)

    zeros16 = jnp.zeros((C,), jnp.float32)
    iota16 = lax.iota(jnp.int32, C)

    for q in range(CONF_WORDS // C):
        conf_v[pl.ds(q * C, C)] = zeros16

    def issue(b, ck):
        start = ck * R
        pltpu.async_copy(x_hbm.at[:, pl.ds(start, R)], xb[b], sx[b])
        pltpu.async_copy(t_hbm.at[pl.ds(start, R)], tb[b], st[b])
        pltpu.async_copy(w_hbm.at[pl.ds(start, R)], wb[b], sw[b])

    def wait(b):
        pltpu.make_async_copy(x_hbm.at[:, pl.ds(0, R)], xb[b], sx[b]).wait()
        pltpu.make_async_copy(t_hbm.at[pl.ds(0, R)], tb[b], st[b]).wait()
        pltpu.make_async_copy(w_hbm.at[pl.ds(0, R)], wb[b], sw[b]).wait()

    def flush(psums, ct):
        for j in range(C):
            plsc.addupdate_scatter(
                conf_v, [ct * (C * C) + j * C + iota16], psums[j])

    def process(b):
        t_ref, w_ref, x_ref = tb[b], wb[b], xb[b]
        first = t_ref[pl.ds(0, C)][0]
        last = t_ref[pl.ds(R - C, C)][15]

        @pl.when(first == last)
        def _uniform_chunk():
            def group(i, psums):
                psums = list(psums)
                for u in range(UNROLL):
                    base = (i * UNROLL + u) * C
                    wvec = w_ref[pl.ds(base, C)]
                    for j in range(C):
                        psums[j] = psums[j] + wvec * x_ref[j, pl.ds(base, C)]
                return tuple(psums)

            psums = lax.fori_loop(0, BLK // UNROLL, group, (zeros16,) * C)
            flush(list(psums), first)

        @pl.when(first != last)
        def _boundary_chunk():
            def group(i, carry):
                ct = carry[-1]
                psums = list(carry[:C])
                base = i * C
                tvec = t_ref[pl.ds(base, C)]
                wvec = w_ref[pl.ds(base, C)]
                wm = jnp.where(tvec == ct, wvec, 0.0)
                xvs = [x_ref[j, pl.ds(base, C)] for j in range(C)]
                psums = [psums[j] + wm * xvs[j] for j in range(C)]

                def boundary(ct):
                    flush(psums, ct)
                    wleft = wvec - wm
                    for j in range(C):
                        plsc.addupdate_scatter(
                            conf_v,
                            [tvec * (C * C) + j * C + iota16],
                            wleft * xvs[j])
                    return tvec[15], jnp.float32(0.0)

                def same(ct):
                    return ct, jnp.float32(1.0)

                ct, keep = lax.cond(tvec[15] != ct, boundary, same, ct)
                return (*[p * keep for p in psums], ct)

            carry = lax.fori_loop(0, BLK, group, ((zeros16,) * C) + (first,))
            flush(list(carry[:C]), carry[-1])

    issue(0, wid)
    issue(1, wid + NW)
    issue(2, wid + 2 * NW)

    def triple(p, _):
        for b in range(3):
            k = 3 * p + b

            @pl.when(k < nchunks)
            def _():
                wait(b)
                process(b)

                @pl.when(k + 3 < nchunks)
                def _():
                    issue(b, wid + NW * (k + 3))

        return 0

    lax.fori_loop(0, MAX_TRIPLES, triple, 0)
    pltpu.sync_copy(conf_v, out_hbm.at[pl.ds(wid * CONF_WORDS, CONF_WORDS)])


def _sc_confusion(xT, t, w):
    mesh = plsc.VectorSubcoreMesh(core_axis_name="c", subcore_axis_name="s")
    return pl.kernel(
        _sc_body,
        out_type=jax.ShapeDtypeStruct((NW * CONF_WORDS,), jnp.float32),
        mesh=mesh,
        compiler_params=pltpu.CompilerParams(needs_layout_passes=False),
        scratch_types=(
            [pltpu.VMEM((C, R), jnp.float32)] * 3
            + [pltpu.VMEM((R,), jnp.int32)] * 3
            + [pltpu.VMEM((R,), jnp.float32)] * 3
            + [pltpu.VMEM((CONF_WORDS,), jnp.float32)]
            + [pltpu.SemaphoreType.DMA] * 9
        ),
    )(xT, t, w)


def _epi_body(p_ref, xt_ref, t_ref, w_ref, o_ref):
    nvalid = TAIL
    t2 = t_ref[...].reshape(1, TAIL_BLK)
    w2 = w_ref[...].reshape(1, TAIL_BLK)
    lane = lax.broadcasted_iota(jnp.int32, (1, TAIL_BLK), 1)
    wv = jnp.where(lane < nvalid, w2, 0.0)
    xt = xt_ref[...]
    tail_jc = jnp.zeros((C, C), jnp.float32)
    for c in range(C):
        wc = jnp.where(t2 == c, wv, 0.0)
        s_c = jnp.sum(xt * wc, axis=1, keepdims=True)
        cl = lax.broadcasted_iota(jnp.int32, (C, C), 1)
        tail_jc = jnp.where(cl == c, s_c, tail_jc)

    conf = jnp.sum(p_ref[...], axis=(0, 3)) + tail_jc.T
    eps = jnp.float32(jnp.finfo(jnp.float32).eps)
    conf = conf * ((1.0 - eps * C) / jnp.sum(conf))
    r = lax.broadcasted_iota(jnp.int32, (C, C), 0)
    c = lax.broadcasted_iota(jnp.int32, (C, C), 1)
    eye = (r == c).astype(jnp.float32)
    conf = conf + eye * eps
    cols = jnp.sum(conf, axis=0, keepdims=True)
    rows = jnp.sum(conf.T, axis=0, keepdims=True)
    diag = jnp.sum(conf * eye, axis=0, keepdims=True)
    kap = 2.0 * (diag - cols * rows) / (cols + rows - 2.0 * cols * rows)
    tr = jnp.float32(0.066)
    kap = jnp.where(kap >= tr, kap, tr * (jnp.tanh((kap - tr) / tr) + 1.0))
    kap = jnp.where(kap > 0, kap, eps)
    o_ref[0, 0] = 1.0 - jnp.exp(jnp.sum(jnp.log(kap)) / C)


def _epilogue(partials, xT, t, w):
    tail_idx = (NCHUNKS * R) // TAIL_BLK
    return pl.pallas_call(
        _epi_body,
        grid=(1,),
        in_specs=[
            pl.BlockSpec((NW, C, C, C), lambda i: (0, 0, 0, 0)),
            pl.BlockSpec((C, TAIL_BLK), lambda i: (0, tail_idx)),
            pl.BlockSpec((TAIL_BLK,), lambda i: (tail_idx,)),
            pl.BlockSpec((TAIL_BLK,), lambda i: (tail_idx,)),
        ],
        out_specs=pl.BlockSpec(
            block_shape=(1, 1), index_map=lambda i: (0, 0),
            memory_space=pltpu.SMEM,
        ),
        out_shape=jax.ShapeDtypeStruct((1, 1), jnp.float32),
    )(partials, xT, t, w)


@jax.jit
def kernel(input, target_classes, weights):
    xT = input.T
    partials = _sc_confusion(xT, target_classes, weights)
    return _epilogue(
        partials.reshape(NW, C, C, C), xT, target_classes, weights
    ).reshape(())


# speedup vs baseline: 52.1708x; 1.0094x over previous
import functools

import jax
import jax.numpy as jnp
from jax import lax
from jax.experimental import pallas as pl
from jax.experimental.pallas import tpu as pltpu
from jax.experimental.pallas import tpu_sc as plsc

C = 16
N = 1_000_000
R = 2_048
NCHUNKS = N // R
TAIL = N - NCHUNKS * R
NW = 32
BLK = R // 16
MAX_TRIPLES = (NCHUNKS // NW + 3) // 3 + 1
UNROLL = 4
CONF_WORDS = C * C * C
TAIL_BLK = 1024


def _sc_body(x_hbm, t_hbm, w_hbm, out_hbm,
             xb0, xb1, xb2, tb0, tb1, tb2, wb0, wb1, wb2, conf_v, tfull,
             sx0, sx1, sx2, st0, st1, st2, sw0, sw1, sw2):
    cid = lax.axis_index("c")
    sid = lax.axis_index("s")
    wid = sid * 2 + cid
    nchunks = jnp.int32(NCHUNKS // NW) + (wid < NCHUNKS % NW).astype(jnp.int32)

    xb = (xb0, xb1, xb2)
    tb = (tb0, tb1, tb2)
    wb = (wb0, wb1, wb2)
    sx = (sx0, sx1, sx2)
    st = (st0, st1, st2)
    sw = (sw0, sw1, sw2)

    zeros16 = jnp.zeros((C,), jnp.float32)
    iota16 = lax.iota(jnp.int32, C)

    for q in range(CONF_WORDS // C):
        conf_v[pl.ds(q * C, C)] = zeros16

    def issue(b, ck):
        start = ck * R
        pltpu.async_copy(x_hbm.at[:, pl.ds(start, R)], xb[b], sx[b])
        pltpu.async_copy(
            t_hbm.at[pl.ds(start, C)], tb[b].at[pl.ds(0, C)], st[b])
        pltpu.async_copy(
            t_hbm.at[pl.ds(start + R - C, C)], tb[b].at[pl.ds(C, C)], st[b])
        pltpu.async_copy(w_hbm.at[pl.ds(start, R)], wb[b], sw[b])

    def wait(b):
        pltpu.make_async_copy(x_hbm.at[:, pl.ds(0, R)], xb[b], sx[b]).wait()
        pltpu.make_async_copy(
            t_hbm.at[pl.ds(0, C)], tb[b].at[pl.ds(0, C)], st[b]).wait()
        pltpu.make_async_copy(
            t_hbm.at[pl.ds(0, C)], tb[b].at[pl.ds(C, C)], st[b]).wait()
        pltpu.make_async_copy(w_hbm.at[pl.ds(0, R)], wb[b], sw[b]).wait()

    def flush(psums, ct):
        for j in range(C):
            plsc.addupdate_scatter(
                conf_v, [ct * (C * C) + j * C + iota16], psums[j])

    def process(b, start):
        w_ref, x_ref = wb[b], xb[b]
        first = tb[b][pl.ds(0, C)][0]
        last = tb[b][pl.ds(C, C)][15]

        @pl.when(first == last)
        def _uniform_chunk():
            def group(i, psums):
                psums = list(psums)
                for u in range(UNROLL):
                    base = (i * UNROLL + u) * C
                    wvec = w_ref[pl.ds(base, C)]
                    for j in range(C):
                        psums[j] = psums[j] + wvec * x_ref[j, pl.ds(base, C)]
                return tuple(psums)

            psums = lax.fori_loop(0, BLK // UNROLL, group, (zeros16,) * C)
            flush(list(psums), first)

        @pl.when(first != last)
        def _boundary_chunk():
            pltpu.sync_copy(t_hbm.at[pl.ds(start, R)], tfull)
            t_ref = tfull

            def group(i, carry):
                ct = carry[-1]
                psums = list(carry[:C])
                base = i * C
                tvec = t_ref[pl.ds(base, C)]
                wvec = w_ref[pl.ds(base, C)]
                wm = jnp.where(tvec == ct, wvec, 0.0)
                xvs = [x_ref[j, pl.ds(base, C)] for j in range(C)]
                psums = [psums[j] + wm * xvs[j] for j in range(C)]

                def boundary(ct):
                    flush(psums, ct)
                    wleft = wvec - wm
                    for j in range(C):
                        plsc.addupdate_scatter(
                            conf_v,
                            [tvec * (C * C) + j * C + iota16],
                            wleft * xvs[j])
                    return tvec[15], jnp.float32(0.0)

                def same(ct):
                    return ct, jnp.float32(1.0)

                ct, keep = lax.cond(tvec[15] != ct, boundary, same, ct)
                return (*[p * keep for p in psums], ct)

            carry = lax.fori_loop(0, BLK, group, ((zeros16,) * C) + (first,))
            flush(list(carry[:C]), carry[-1])

    issue(0, wid)
    issue(1, wid + NW)
    issue(2, wid + 2 * NW)

    def triple(p, _):
        for b in range(3):
            k = 3 * p + b

            @pl.when(k < nchunks)
            def _():
                wait(b)
                process(b, (wid + NW * k) * R)

                @pl.when(k + 3 < nchunks)
                def _():
                    issue(b, wid + NW * (k + 3))

        return 0

    lax.fori_loop(0, MAX_TRIPLES, triple, 0)
    pltpu.sync_copy(conf_v, out_hbm.at[pl.ds(wid * CONF_WORDS, CONF_WORDS)])


def _sc_confusion(xT, t, w):
    mesh = plsc.VectorSubcoreMesh(core_axis_name="c", subcore_axis_name="s")
    return pl.kernel(
        _sc_body,
        out_type=jax.ShapeDtypeStruct((NW * CONF_WORDS,), jnp.float32),
        mesh=mesh,
        compiler_params=pltpu.CompilerParams(needs_layout_passes=False),
        scratch_types=(
            [pltpu.VMEM((C, R), jnp.float32)] * 3
            + [pltpu.VMEM((2 * C,), jnp.int32)] * 3
            + [pltpu.VMEM((R,), jnp.float32)] * 3
            + [pltpu.VMEM((CONF_WORDS,), jnp.float32)]
            + [pltpu.VMEM((R,), jnp.int32)]
            + [pltpu.SemaphoreType.DMA] * 9
        ),
    )(xT, t, w)


def _epi_body(p_ref, xt_ref, t_ref, w_ref, o_ref):
    nvalid = TAIL
    t2 = t_ref[...].reshape(1, TAIL_BLK)
    w2 = w_ref[...].reshape(1, TAIL_BLK)
    lane = lax.broadcasted_iota(jnp.int32, (1, TAIL_BLK), 1)
    wv = jnp.where(lane < nvalid, w2, 0.0)
    xt = xt_ref[...]
    tail_jc = jnp.zeros((C, C), jnp.float32)
    for c in range(C):
        wc = jnp.where(t2 == c, wv, 0.0)
        s_c = jnp.sum(xt * wc, axis=1, keepdims=True)
        cl = lax.broadcasted_iota(jnp.int32, (C, C), 1)
        tail_jc = jnp.where(cl == c, s_c, tail_jc)

    conf = jnp.sum(p_ref[...], axis=(0, 3)) + tail_jc.T
    eps = jnp.float32(jnp.finfo(jnp.float32).eps)
    conf = conf * ((1.0 - eps * C) / jnp.sum(conf))
    r = lax.broadcasted_iota(jnp.int32, (C, C), 0)
    c = lax.broadcasted_iota(jnp.int32, (C, C), 1)
    eye = (r == c).astype(jnp.float32)
    conf = conf + eye * eps
    cols = jnp.sum(conf, axis=0, keepdims=True)
    rows = jnp.sum(conf.T, axis=0, keepdims=True)
    diag = jnp.sum(conf * eye, axis=0, keepdims=True)
    kap = 2.0 * (diag - cols * rows) / (cols + rows - 2.0 * cols * rows)
    tr = jnp.float32(0.066)
    kap = jnp.where(kap >= tr, kap, tr * (jnp.tanh((kap - tr) / tr) + 1.0))
    kap = jnp.where(kap > 0, kap, eps)
    o_ref[0, 0] = 1.0 - jnp.exp(jnp.sum(jnp.log(kap)) / C)


def _epilogue(partials, xT, t, w):
    tail_idx = (NCHUNKS * R) // TAIL_BLK
    return pl.pallas_call(
        _epi_body,
        grid=(1,),
        in_specs=[
            pl.BlockSpec((NW, C, C, C), lambda i: (0, 0, 0, 0)),
            pl.BlockSpec((C, TAIL_BLK), lambda i: (0, tail_idx)),
            pl.BlockSpec((TAIL_BLK,), lambda i: (tail_idx,)),
            pl.BlockSpec((TAIL_BLK,), lambda i: (tail_idx,)),
        ],
        out_specs=pl.BlockSpec(
            block_shape=(1, 1), index_map=lambda i: (0, 0),
            memory_space=pltpu.SMEM,
        ),
        out_shape=jax.ShapeDtypeStruct((1, 1), jnp.float32),
    )(partials, xT, t, w)


@jax.jit
def kernel(input, target_classes, weights):
    xT = input.T
    partials = _sc_confusion(xT, target_classes, weights)
    return _epilogue(
        partials.reshape(NW, C, C, C), xT, target_classes, weights
    ).reshape(())


# speedup vs baseline: 54.2510x; 1.0399x over previous
import functools

import jax
import jax.numpy as jnp
from jax import lax
from jax.experimental import pallas as pl
from jax.experimental.pallas import tpu as pltpu
from jax.experimental.pallas import tpu_sc as plsc

C = 16
N = 1_000_000
R = 2_048
BT = 8_192
TC_ROWS = 262_144
NFULL = N // R * R
TAIL = N - NFULL
SC_CK0 = TC_ROWS // R
NCHUNKS = (NFULL - TC_ROWS) // R
NW = 32
BLK = R // 16
MAX_TRIPLES = (NCHUNKS // NW + 3) // 3 + 1
UNROLL = 4
CONF_WORDS = C * C * C
TAIL_BLK = 1024
TAIL_IDX = NFULL // TAIL_BLK


def _sc_body(x_hbm, t_hbm, w_hbm, out_hbm,
             xb0, xb1, xb2, tb0, tb1, tb2, wb0, wb1, wb2, conf_v, tfull,
             sx0, sx1, sx2, st0, st1, st2, sw0, sw1, sw2):
    cid = lax.axis_index("c")
    sid = lax.axis_index("s")
    wid = sid * 2 + cid
    nchunks = jnp.int32(NCHUNKS // NW) + (wid < NCHUNKS % NW).astype(jnp.int32)

    xb = (xb0, xb1, xb2)
    tb = (tb0, tb1, tb2)
    wb = (wb0, wb1, wb2)
    sx = (sx0, sx1, sx2)
    st = (st0, st1, st2)
    sw = (sw0, sw1, sw2)

    zeros16 = jnp.zeros((C,), jnp.float32)
    iota16 = lax.iota(jnp.int32, C)

    for q in range(CONF_WORDS // C):
        conf_v[pl.ds(q * C, C)] = zeros16

    def issue(b, ck):
        start = ck * R
        pltpu.async_copy(x_hbm.at[:, pl.ds(start, R)], xb[b], sx[b])
        pltpu.async_copy(
            t_hbm.at[pl.ds(start, C)], tb[b].at[pl.ds(0, C)], st[b])
        pltpu.async_copy(
            t_hbm.at[pl.ds(start + R - C, C)], tb[b].at[pl.ds(C, C)], st[b])
        pltpu.async_copy(w_hbm.at[pl.ds(start, R)], wb[b], sw[b])

    def wait(b):
        pltpu.make_async_copy(x_hbm.at[:, pl.ds(0, R)], xb[b], sx[b]).wait()
        pltpu.make_async_copy(
            t_hbm.at[pl.ds(0, C)], tb[b].at[pl.ds(0, C)], st[b]).wait()
        pltpu.make_async_copy(
            t_hbm.at[pl.ds(0, C)], tb[b].at[pl.ds(C, C)], st[b]).wait()
        pltpu.make_async_copy(w_hbm.at[pl.ds(0, R)], wb[b], sw[b]).wait()

    def flush(psums, ct):
        for j in range(C):
            plsc.addupdate_scatter(
                conf_v, [ct * (C * C) + j * C + iota16], psums[j])

    def process(b, start):
        w_ref, x_ref = wb[b], xb[b]
        first = tb[b][pl.ds(0, C)][0]
        last = tb[b][pl.ds(C, C)][15]

        @pl.when(first == last)
        def _uniform_chunk():
            def group(i, psums):
                psums = list(psums)
                for u in range(UNROLL):
                    base = (i * UNROLL + u) * C
                    wvec = w_ref[pl.ds(base, C)]
                    for j in range(C):
                        psums[j] = psums[j] + wvec * x_ref[j, pl.ds(base, C)]
                return tuple(psums)

            psums = lax.fori_loop(0, BLK // UNROLL, group, (zeros16,) * C)
            flush(list(psums), first)

        @pl.when(first != last)
        def _boundary_chunk():
            pltpu.sync_copy(t_hbm.at[pl.ds(start, R)], tfull)
            t_ref = tfull

            def group(i, carry):
                ct = carry[-1]
                psums = list(carry[:C])
                base = i * C
                tvec = t_ref[pl.ds(base, C)]
                wvec = w_ref[pl.ds(base, C)]
                wm = jnp.where(tvec == ct, wvec, 0.0)
                xvs = [x_ref[j, pl.ds(base, C)] for j in range(C)]
                psums = [psums[j] + wm * xvs[j] for j in range(C)]

                def boundary(ct):
                    flush(psums, ct)
                    wleft = wvec - wm
                    for j in range(C):
                        plsc.addupdate_scatter(
                            conf_v,
                            [tvec * (C * C) + j * C + iota16],
                            wleft * xvs[j])
                    return tvec[15], jnp.float32(0.0)

                def same(ct):
                    return ct, jnp.float32(1.0)

                ct, keep = lax.cond(tvec[15] != ct, boundary, same, ct)
                return (*[p * keep for p in psums], ct)

            carry = lax.fori_loop(0, BLK, group, ((zeros16,) * C) + (first,))
            flush(list(carry[:C]), carry[-1])

    issue(0, SC_CK0 + wid)
    issue(1, SC_CK0 + wid + NW)
    issue(2, SC_CK0 + wid + 2 * NW)

    def triple(p, _):
        for b in range(3):
            k = 3 * p + b

            @pl.when(k < nchunks)
            def _():
                wait(b)
                process(b, (SC_CK0 + wid + NW * k) * R)

                @pl.when(k + 3 < nchunks)
                def _():
                    issue(b, SC_CK0 + wid + NW * (k + 3))

        return 0

    lax.fori_loop(0, MAX_TRIPLES, triple, 0)
    pltpu.sync_copy(conf_v, out_hbm.at[pl.ds(wid * CONF_WORDS, CONF_WORDS)])


def _sc_confusion(xT, t, w):
    mesh = plsc.VectorSubcoreMesh(core_axis_name="c", subcore_axis_name="s")
    return pl.kernel(
        _sc_body,
        out_type=jax.ShapeDtypeStruct((NW * CONF_WORDS,), jnp.float32),
        mesh=mesh,
        compiler_params=pltpu.CompilerParams(needs_layout_passes=False),
        scratch_types=(
            [pltpu.VMEM((C, R), jnp.float32)] * 3
            + [pltpu.VMEM((2 * C,), jnp.int32)] * 3
            + [pltpu.VMEM((R,), jnp.float32)] * 3
            + [pltpu.VMEM((CONF_WORDS,), jnp.float32)]
            + [pltpu.VMEM((R,), jnp.int32)]
            + [pltpu.SemaphoreType.DMA] * 9
        ),
    )(xT, t, w)


def _class_accum(acc_jc, xt, t2, wv):
    cl = lax.broadcasted_iota(jnp.int32, (C, C), 1)
    for c in range(C):
        wc = jnp.where(t2 == c, wv, 0.0)
        s_c = jnp.sum(xt * wc, axis=1, keepdims=True)
        acc_jc = acc_jc + jnp.where(cl == c, s_c, 0.0)
    return acc_jc


def _bulk_body(xt_ref, t_ref, w_ref, xtt_ref, tt_ref, wt_ref, o_ref, acc_ref):
    i = pl.program_id(0)

    @pl.when(i == 0)
    def _init():
        acc_ref[...] = jnp.zeros((C, C), jnp.float32)

    t = t_ref[...]
    w2 = w_ref[...].reshape(1, BT)
    xt = xt_ref[...]
    first = t[0]
    last = t[BT - 1]

    @pl.when(first == last)
    def _uniform():
        s = jnp.sum(xt * w2, axis=1, keepdims=True)
        cl = lax.broadcasted_iota(jnp.int32, (C, C), 1)
        acc_ref[...] += jnp.where(cl == first, s, 0.0)

    @pl.when(first != last)
    def _boundary():
        acc_ref[...] = _class_accum(
            acc_ref[...], xt, t.reshape(1, BT), w2)

    @pl.when(i == pl.num_programs(0) - 1)
    def _tail():
        t2 = tt_ref[...].reshape(1, TAIL_BLK)
        w2t = wt_ref[...].reshape(1, TAIL_BLK)
        lane = lax.broadcasted_iota(jnp.int32, (1, TAIL_BLK), 1)
        wv = jnp.where(lane < TAIL, w2t, 0.0)
        o_ref[...] = _class_accum(acc_ref[...], xtt_ref[...], t2, wv)


def _tc_bulk(xT, t, w):
    return pl.pallas_call(
        _bulk_body,
        grid=(TC_ROWS // BT,),
        in_specs=[
            pl.BlockSpec((C, BT), lambda i: (0, i)),
            pl.BlockSpec((BT,), lambda i: (i,)),
            pl.BlockSpec((BT,), lambda i: (i,)),
            pl.BlockSpec((C, TAIL_BLK), lambda i: (0, TAIL_IDX)),
            pl.BlockSpec((TAIL_BLK,), lambda i: (TAIL_IDX,)),
            pl.BlockSpec((TAIL_BLK,), lambda i: (TAIL_IDX,)),
        ],
        out_specs=pl.BlockSpec((C, C), lambda i: (0, 0)),
        out_shape=jax.ShapeDtypeStruct((C, C), jnp.float32),
        scratch_shapes=[pltpu.VMEM((C, C), jnp.float32)],
    )(xT, t, w, xT, t, w)


def _epi_body(p_ref, b_ref, o_ref):
    conf = jnp.sum(p_ref[...], axis=(0, 3)) + b_ref[...].T
    eps = jnp.float32(jnp.finfo(jnp.float32).eps)
    conf = conf * ((1.0 - eps * C) / jnp.sum(conf))
    r = lax.broadcasted_iota(jnp.int32, (C, C), 0)
    c = lax.broadcasted_iota(jnp.int32, (C, C), 1)
    eye = (r == c).astype(jnp.float32)
    conf = conf + eye * eps
    cols = jnp.sum(conf, axis=0, keepdims=True)
    rows = jnp.sum(conf.T, axis=0, keepdims=True)
    diag = jnp.sum(conf * eye, axis=0, keepdims=True)
    kap = 2.0 * (diag - cols * rows) / (cols + rows - 2.0 * cols * rows)
    tr = jnp.float32(0.066)
    kap = jnp.where(kap >= tr, kap, tr * (jnp.tanh((kap - tr) / tr) + 1.0))
    kap = jnp.where(kap > 0, kap, eps)
    o_ref[0, 0] = 1.0 - jnp.exp(jnp.sum(jnp.log(kap)) / C)


def _epilogue(partials, bulk):
    return pl.pallas_call(
        _epi_body,
        out_specs=pl.BlockSpec(memory_space=pltpu.SMEM),
        out_shape=jax.ShapeDtypeStruct((1, 1), jnp.float32),
    )(partials, bulk)


@jax.jit
def kernel(input, target_classes, weights):
    xT = input.T
    partials = _sc_confusion(xT, target_classes, weights)
    bulk = _tc_bulk(xT, target_classes, weights)
    return _epilogue(partials.reshape(NW, C, C, C), bulk).reshape(())
